# baseline (speedup 1.0000x reference)
import jax
import jax.numpy as jnp
from jax.experimental import pallas as pl
from jax.experimental.pallas import tpu as pltpu


def _gate_kernel(xa_ref, xb_ref, za_ref, zb_ref, w_ref, b_ref, out_ref):
    f32 = jnp.float32
    dx = xa_ref.shape[1]
    half = xa_ref.shape[0]
    for x_ref, z_ref, rows in (
        (xa_ref, za_ref, slice(0, half)),
        (xb_ref, zb_ref, slice(half, 2 * half)),
    ):
        p = jnp.dot(x_ref[...], w_ref[:dx, :], preferred_element_type=f32)
        p += jnp.dot(z_ref[...], w_ref[dx:, :], preferred_element_type=f32)
        logits = p + b_ref[...]
        m = jnp.max(logits, axis=1, keepdims=True)
        e = jnp.exp(logits - m)
        out_ref[rows, :] = e / jnp.sum(e, axis=1, keepdims=True)


def kernel(x, z, W, b):
    n_tokens, dx = x.shape
    dz = z.shape[1]
    k, num_experts = W.shape

    block = 512
    half = block // 2
    grid = (n_tokens // block,)

    return pl.pallas_call(
        _gate_kernel,
        grid=grid,
        in_specs=[
            pl.BlockSpec((half, dx), lambda i: (2 * i, 0)),
            pl.BlockSpec((half, dx), lambda i: (2 * i + 1, 0)),
            pl.BlockSpec((half, dz), lambda i: (2 * i, 0)),
            pl.BlockSpec((half, dz), lambda i: (2 * i + 1, 0)),
            pl.BlockSpec((k, num_experts), lambda i: (0, 0)),
            pl.BlockSpec((1, num_experts), lambda i: (0, 0)),
        ],
        out_specs=pl.BlockSpec((block, num_experts), lambda i: (i, 0)),
        out_shape=jax.ShapeDtypeStruct((n_tokens, num_experts), jnp.float32),
        compiler_params=pltpu.CompilerParams(
            dimension_semantics=("parallel",),
        ),
    )(x, x, z, z, W, b.reshape(1, num_experts))
